# baseline (speedup 1.0000x reference)
import jax
import jax.numpy as jnp
from jax.experimental import pallas as pl
from jax.experimental.pallas import tpu as pltpu

_EPS = 1e-5


def kernel(xyz1, xyz2, feats1, feats2, W1, b1, g1, be1, W2, b2, g2, be2):
    B, N1, _ = xyz1.shape
    N2 = xyz2.shape[1]
    C1 = feats1.shape[1]
    C2 = feats2.shape[1]
    OC1 = W1.shape[0]
    OC2 = W2.shape[0]
    TQ = 512
    NT = N1 // TQ
    NPTS = B * N1

    W1a = W1[:, :C2]
    W1b = W1[:, C2:]
    W2b = W2.astype(jnp.bfloat16)
    xyz1c = xyz1 - 0.5
    xyz2t = jnp.transpose(xyz2, (0, 2, 1)) - 0.5

    def ka(xyz1_ref, xyz2t_ref, f2_ref, w1a_ref, f1_ref, w1b_ref,
           x1_ref, st_ref, g_scr):
        b = pl.program_id(0)
        t = pl.program_id(1)

        @pl.when(t == 0)
        def _():
            g_scr[...] = jax.lax.dot_general(
                f2_ref[0], w1a_ref[...], (((0,), (1,)), ((), ())),
                preferred_element_type=jnp.float32).astype(jnp.bfloat16)

        q = xyz1_ref[0]
        k2 = xyz2t_ref[0]
        qn = jnp.sum(q * q, axis=1, keepdims=True)
        kn = jnp.sum(k2 * k2, axis=0, keepdims=True)
        qk = jax.lax.dot_general(
            q, k2, (((1,), (0,)), ((), ())),
            preferred_element_type=jnp.float32,
            precision=jax.lax.Precision.HIGHEST)
        ss = (qn + kn) - 2.0 * qk

        inf = jnp.float32(jnp.inf)
        zero = jnp.float32(0.0)
        dm = ss
        Wd = None
        sw = None
        for kpass in range(3):
            mk = jnp.min(dm, axis=1, keepdims=True)
            mask = dm == mk
            if kpass < 2:
                dm = jnp.where(mask, inf, dm)
            dk = jnp.maximum(jnp.sqrt(jnp.maximum(mk, 1e-16)), 1e-8)
            wk = 1.0 / dk
            contrib = jnp.where(mask, wk, zero)
            Wd = contrib if Wd is None else Wd + contrib
            sw = wk if sw is None else sw + wk

        x1 = jax.lax.dot_general(
            Wd.astype(jnp.bfloat16), g_scr[...], (((1,), (0,)), ((), ())),
            preferred_element_type=jnp.float32)
        x1 = x1 * (1.0 / sw)
        x1 = x1 + jax.lax.dot_general(
            f1_ref[0], w1b_ref[...], (((0,), (1,)), ((), ())),
            preferred_element_type=jnp.float32)
        x1_ref[0] = x1.astype(jnp.bfloat16)

        @pl.when(jnp.logical_and(b == 0, t == 0))
        def _():
            st_ref[...] = jnp.zeros_like(st_ref)

        st_ref[0:1, :] += jnp.sum(x1, axis=0, keepdims=True)
        st_ref[1:2, :] += jnp.sum(x1 * x1, axis=0, keepdims=True)

    x1, stats1 = pl.pallas_call(
        ka,
        grid=(B, NT),
        in_specs=[
            pl.BlockSpec((1, TQ, 3), lambda b, t: (b, t, 0)),
            pl.BlockSpec((1, 3, N2), lambda b, t: (b, 0, 0)),
            pl.BlockSpec((1, C2, N2), lambda b, t: (b, 0, 0)),
            pl.BlockSpec((OC1, C2), lambda b, t: (0, 0)),
            pl.BlockSpec((1, C1, TQ), lambda b, t: (b, 0, t)),
            pl.BlockSpec((OC1, C1), lambda b, t: (0, 0)),
        ],
        out_specs=[
            pl.BlockSpec((1, TQ, OC1), lambda b, t: (b, t, 0)),
            pl.BlockSpec((2, OC1), lambda b, t: (0, 0)),
        ],
        out_shape=[
            jax.ShapeDtypeStruct((B, N1, OC1), jnp.bfloat16),
            jax.ShapeDtypeStruct((2, OC1), jnp.float32),
        ],
        scratch_shapes=[pltpu.VMEM((N2, OC1), jnp.bfloat16)],
    )(xyz1c, xyz2t, feats2, W1a, feats1, W1b)

    mean1 = stats1[0] / NPTS
    var1 = stats1[1] / NPTS - mean1 * mean1
    rstd1 = g1 / jnp.sqrt(var1 + _EPS)
    sc1 = rstd1.reshape(1, OC1)
    sh1 = (be1 - mean1 * rstd1).reshape(1, OC1)

    def kb(x1_ref, sc_ref, sh_ref, w2_ref, x2_ref, st_ref):
        b = pl.program_id(0)
        t = pl.program_id(1)
        x1f = x1_ref[0].astype(jnp.float32)
        r = jnp.maximum(x1f * sc_ref[...] + sh_ref[...], 0.0)
        x2t = jax.lax.dot_general(
            w2_ref[...], r.astype(jnp.bfloat16), (((1,), (1,)), ((), ())),
            preferred_element_type=jnp.float32)
        x2_ref[0] = x2t.astype(jnp.bfloat16)

        @pl.when(jnp.logical_and(b == 0, t == 0))
        def _():
            st_ref[...] = jnp.zeros_like(st_ref)

        st_ref[:, 0:1] += jnp.sum(x2t, axis=1, keepdims=True)
        st_ref[:, 1:2] += jnp.sum(x2t * x2t, axis=1, keepdims=True)

    x2, stats2 = pl.pallas_call(
        kb,
        grid=(B, NT),
        in_specs=[
            pl.BlockSpec((1, TQ, OC1), lambda b, t: (b, t, 0)),
            pl.BlockSpec((1, OC1), lambda b, t: (0, 0)),
            pl.BlockSpec((1, OC1), lambda b, t: (0, 0)),
            pl.BlockSpec((OC2, OC1), lambda b, t: (0, 0)),
        ],
        out_specs=[
            pl.BlockSpec((1, OC2, TQ), lambda b, t: (b, 0, t)),
            pl.BlockSpec((OC2, 2), lambda b, t: (0, 0)),
        ],
        out_shape=[
            jax.ShapeDtypeStruct((B, OC2, N1), jnp.bfloat16),
            jax.ShapeDtypeStruct((OC2, 2), jnp.float32),
        ],
    )(x1, sc1, sh1, W2b)

    mean2 = stats2[:, 0] / NPTS
    var2 = stats2[:, 1] / NPTS - mean2 * mean2
    rstd2 = g2 / jnp.sqrt(var2 + _EPS)
    sc2 = rstd2.reshape(OC2, 1)
    sh2 = (be2 - mean2 * rstd2).reshape(OC2, 1)

    def kc(x2_ref, sc_ref, sh_ref, o_ref):
        o_ref[0] = jnp.maximum(
            x2_ref[0].astype(jnp.float32) * sc_ref[...] + sh_ref[...], 0.0)

    out = pl.pallas_call(
        kc,
        grid=(B, NT),
        in_specs=[
            pl.BlockSpec((1, OC2, TQ), lambda b, t: (b, 0, t)),
            pl.BlockSpec((OC2, 1), lambda b, t: (0, 0)),
            pl.BlockSpec((OC2, 1), lambda b, t: (0, 0)),
        ],
        out_specs=pl.BlockSpec((1, OC2, TQ), lambda b, t: (b, 0, t)),
        out_shape=jax.ShapeDtypeStruct((B, OC2, N1), jnp.float32),
    )(x2, sc2, sh2)

    return out


# speedup vs baseline: 1.5449x; 1.5449x over previous
import jax
import jax.numpy as jnp
from jax.experimental import pallas as pl
from jax.experimental.pallas import tpu as pltpu

_EPS = 1e-5


def kernel(xyz1, xyz2, feats1, feats2, W1, b1, g1, be1, W2, b2, g2, be2):
    B, N1, _ = xyz1.shape
    N2 = xyz2.shape[1]
    C1 = feats1.shape[1]
    C2 = feats2.shape[1]
    OC1 = W1.shape[0]
    OC2 = W2.shape[0]
    TQ = 1024
    NT = N1 // TQ
    NPTS = B * N1

    W1a = W1[:, :C2]
    W1b = W1[:, C2:]
    W2b = W2.astype(jnp.bfloat16)
    xyz1c = xyz1 - 0.5
    xyz2t = jnp.transpose(xyz2, (0, 2, 1)) - 0.5

    def ka(xyz1_ref, xyz2t_ref, f2_ref, w1a_ref, f1_ref, w1b_ref,
           x1_ref, st_ref, g_scr):
        b = pl.program_id(0)
        t = pl.program_id(1)

        @pl.when(t == 0)
        def _():
            g_scr[...] = jax.lax.dot_general(
                f2_ref[0], w1a_ref[...], (((0,), (1,)), ((), ())),
                preferred_element_type=jnp.float32).astype(jnp.bfloat16)

        q = xyz1_ref[0]
        k2 = xyz2t_ref[0]
        ss = None
        for c in range(3):
            df = q[:, c:c + 1] - k2[c:c + 1, :]
            ss = df * df if ss is None else ss + df * df

        inf = jnp.float32(jnp.inf)
        zero = jnp.float32(0.0)
        dm = ss
        Wd = None
        sw = None
        for kpass in range(3):
            mk = jnp.min(dm, axis=1, keepdims=True)
            mask = dm == mk
            if kpass < 2:
                dm = jnp.where(mask, inf, dm)
            dk = jnp.maximum(jnp.sqrt(jnp.maximum(mk, 1e-16)), 1e-8)
            wk = 1.0 / dk
            contrib = jnp.where(mask, wk, zero)
            Wd = contrib if Wd is None else Wd + contrib
            sw = wk if sw is None else sw + wk

        x1 = jax.lax.dot_general(
            Wd.astype(jnp.bfloat16), g_scr[...], (((1,), (0,)), ((), ())),
            preferred_element_type=jnp.float32)
        x1 = x1 * (1.0 / sw)
        x1 = x1 + jax.lax.dot_general(
            f1_ref[0], w1b_ref[...], (((0,), (1,)), ((), ())),
            preferred_element_type=jnp.float32)
        x1_ref[0] = x1.astype(jnp.bfloat16)

        @pl.when(jnp.logical_and(b == 0, t == 0))
        def _():
            st_ref[...] = jnp.zeros_like(st_ref)

        st_ref[0:1, :] += jnp.sum(x1, axis=0, keepdims=True)
        st_ref[1:2, :] += jnp.sum(x1 * x1, axis=0, keepdims=True)

    x1, stats1 = pl.pallas_call(
        ka,
        grid=(B, NT),
        in_specs=[
            pl.BlockSpec((1, TQ, 3), lambda b, t: (b, t, 0)),
            pl.BlockSpec((1, 3, N2), lambda b, t: (b, 0, 0)),
            pl.BlockSpec((1, C2, N2), lambda b, t: (b, 0, 0)),
            pl.BlockSpec((OC1, C2), lambda b, t: (0, 0)),
            pl.BlockSpec((1, C1, TQ), lambda b, t: (b, 0, t)),
            pl.BlockSpec((OC1, C1), lambda b, t: (0, 0)),
        ],
        out_specs=[
            pl.BlockSpec((1, TQ, OC1), lambda b, t: (b, t, 0)),
            pl.BlockSpec((2, OC1), lambda b, t: (0, 0)),
        ],
        out_shape=[
            jax.ShapeDtypeStruct((B, N1, OC1), jnp.bfloat16),
            jax.ShapeDtypeStruct((2, OC1), jnp.float32),
        ],
        scratch_shapes=[pltpu.VMEM((N2, OC1), jnp.bfloat16)],
    )(xyz1c, xyz2t, feats2, W1a, feats1, W1b)

    mean1 = stats1[0] / NPTS
    var1 = stats1[1] / NPTS - mean1 * mean1
    rstd1 = g1 / jnp.sqrt(var1 + _EPS)
    sc1 = rstd1.reshape(1, OC1)
    sh1 = (be1 - mean1 * rstd1).reshape(1, OC1)

    def kb(x1_ref, sc_ref, sh_ref, w2_ref, x2_ref, st_ref):
        b = pl.program_id(0)
        t = pl.program_id(1)
        x1f = x1_ref[0].astype(jnp.float32)
        r = jnp.maximum(x1f * sc_ref[...] + sh_ref[...], 0.0)
        x2t = jax.lax.dot_general(
            w2_ref[...], r.astype(jnp.bfloat16), (((1,), (1,)), ((), ())),
            preferred_element_type=jnp.float32)
        x2_ref[0] = x2t.astype(jnp.bfloat16)

        @pl.when(jnp.logical_and(b == 0, t == 0))
        def _():
            st_ref[...] = jnp.zeros_like(st_ref)

        st_ref[:, 0:1] += jnp.sum(x2t, axis=1, keepdims=True)
        st_ref[:, 1:2] += jnp.sum(x2t * x2t, axis=1, keepdims=True)

    x2, stats2 = pl.pallas_call(
        kb,
        grid=(B, NT),
        in_specs=[
            pl.BlockSpec((1, TQ, OC1), lambda b, t: (b, t, 0)),
            pl.BlockSpec((1, OC1), lambda b, t: (0, 0)),
            pl.BlockSpec((1, OC1), lambda b, t: (0, 0)),
            pl.BlockSpec((OC2, OC1), lambda b, t: (0, 0)),
        ],
        out_specs=[
            pl.BlockSpec((1, OC2, TQ), lambda b, t: (b, 0, t)),
            pl.BlockSpec((OC2, 2), lambda b, t: (0, 0)),
        ],
        out_shape=[
            jax.ShapeDtypeStruct((B, OC2, N1), jnp.bfloat16),
            jax.ShapeDtypeStruct((OC2, 2), jnp.float32),
        ],
    )(x1, sc1, sh1, W2b)

    mean2 = stats2[:, 0] / NPTS
    var2 = stats2[:, 1] / NPTS - mean2 * mean2
    rstd2 = g2 / jnp.sqrt(var2 + _EPS)
    sc2 = rstd2.reshape(OC2, 1)
    sh2 = (be2 - mean2 * rstd2).reshape(OC2, 1)

    def kc(x2_ref, sc_ref, sh_ref, o_ref):
        o_ref[0] = jnp.maximum(
            x2_ref[0].astype(jnp.float32) * sc_ref[...] + sh_ref[...], 0.0)

    out = pl.pallas_call(
        kc,
        grid=(B, NT),
        in_specs=[
            pl.BlockSpec((1, OC2, TQ), lambda b, t: (b, 0, t)),
            pl.BlockSpec((OC2, 1), lambda b, t: (0, 0)),
            pl.BlockSpec((OC2, 1), lambda b, t: (0, 0)),
        ],
        out_specs=pl.BlockSpec((1, OC2, TQ), lambda b, t: (b, 0, t)),
        out_shape=jax.ShapeDtypeStruct((B, OC2, N1), jnp.float32),
    )(x2, sc2, sh2)

    return out


# speedup vs baseline: 1.6598x; 1.0744x over previous
import jax
import jax.numpy as jnp
from jax.experimental import pallas as pl
from jax.experimental.pallas import tpu as pltpu

_EPS = 1e-5


def kernel(xyz1, xyz2, feats1, feats2, W1, b1, g1, be1, W2, b2, g2, be2):
    B, N1, _ = xyz1.shape
    N2 = xyz2.shape[1]
    C1 = feats1.shape[1]
    C2 = feats2.shape[1]
    OC1 = W1.shape[0]
    OC2 = W2.shape[0]
    TQ = 2048
    NT = N1 // TQ
    NPTS = B * N1

    W1a = W1[:, :C2]
    W1b = W1[:, C2:]
    W2b = W2.astype(jnp.bfloat16)
    xyz1c = xyz1 - 0.5
    xyz2t = jnp.transpose(xyz2, (0, 2, 1)) - 0.5

    def ka(xyz1_ref, xyz2t_ref, f2_ref, w1a_ref, f1_ref, w1b_ref,
           x1_ref, st_ref, g_scr):
        b = pl.program_id(0)
        t = pl.program_id(1)

        @pl.when(t == 0)
        def _():
            g_scr[...] = jax.lax.dot_general(
                f2_ref[0], w1a_ref[...], (((0,), (1,)), ((), ())),
                preferred_element_type=jnp.float32).astype(jnp.bfloat16)

        q = xyz1_ref[0]
        k2 = xyz2t_ref[0]
        ss = None
        for c in range(3):
            df = q[:, c:c + 1] - k2[c:c + 1, :]
            ss = df * df if ss is None else ss + df * df

        inf = jnp.float32(jnp.inf)
        zero = jnp.float32(0.0)
        dm = ss
        Wd = None
        sw = None
        for kpass in range(3):
            mk = jnp.min(dm, axis=1, keepdims=True)
            mask = dm == mk
            if kpass < 2:
                dm = jnp.where(mask, inf, dm)
            dk = jnp.maximum(jnp.sqrt(jnp.maximum(mk, 1e-16)), 1e-8)
            wk = 1.0 / dk
            contrib = jnp.where(mask, wk, zero)
            Wd = contrib if Wd is None else Wd + contrib
            sw = wk if sw is None else sw + wk

        x1 = jax.lax.dot_general(
            Wd.astype(jnp.bfloat16), g_scr[...], (((1,), (0,)), ((), ())),
            preferred_element_type=jnp.float32)
        x1 = x1 * (1.0 / sw)
        x1 = x1 + jax.lax.dot_general(
            f1_ref[0], w1b_ref[...], (((0,), (1,)), ((), ())),
            preferred_element_type=jnp.float32)
        x1_ref[0] = x1.astype(jnp.bfloat16)

        @pl.when(jnp.logical_and(b == 0, t == 0))
        def _():
            st_ref[...] = jnp.zeros_like(st_ref)

        st_ref[0:1, :] += jnp.sum(x1, axis=0, keepdims=True)
        st_ref[1:2, :] += jnp.sum(x1 * x1, axis=0, keepdims=True)

    x1, stats1 = pl.pallas_call(
        ka,
        grid=(B, NT),
        in_specs=[
            pl.BlockSpec((1, TQ, 3), lambda b, t: (b, t, 0)),
            pl.BlockSpec((1, 3, N2), lambda b, t: (b, 0, 0)),
            pl.BlockSpec((1, C2, N2), lambda b, t: (b, 0, 0)),
            pl.BlockSpec((OC1, C2), lambda b, t: (0, 0)),
            pl.BlockSpec((1, C1, TQ), lambda b, t: (b, 0, t)),
            pl.BlockSpec((OC1, C1), lambda b, t: (0, 0)),
        ],
        out_specs=[
            pl.BlockSpec((1, TQ, OC1), lambda b, t: (b, t, 0)),
            pl.BlockSpec((2, OC1), lambda b, t: (0, 0)),
        ],
        out_shape=[
            jax.ShapeDtypeStruct((B, N1, OC1), jnp.bfloat16),
            jax.ShapeDtypeStruct((2, OC1), jnp.float32),
        ],
        scratch_shapes=[pltpu.VMEM((N2, OC1), jnp.bfloat16)],
    )(xyz1c, xyz2t, feats2, W1a, feats1, W1b)

    mean1 = stats1[0] / NPTS
    var1 = stats1[1] / NPTS - mean1 * mean1
    rstd1 = g1 / jnp.sqrt(var1 + _EPS)
    sc1 = rstd1.reshape(1, OC1)
    sh1 = (be1 - mean1 * rstd1).reshape(1, OC1)

    def kb(x1_ref, sc_ref, sh_ref, w2_ref, x2_ref, st_ref):
        b = pl.program_id(0)
        t = pl.program_id(1)
        x1f = x1_ref[0].astype(jnp.float32)
        r = jnp.maximum(x1f * sc_ref[...] + sh_ref[...], 0.0)
        x2t = jax.lax.dot_general(
            w2_ref[...], r.astype(jnp.bfloat16), (((1,), (1,)), ((), ())),
            preferred_element_type=jnp.float32)
        x2_ref[0] = x2t.astype(jnp.bfloat16)

        @pl.when(jnp.logical_and(b == 0, t == 0))
        def _():
            st_ref[...] = jnp.zeros_like(st_ref)

        st_ref[:, 0:1] += jnp.sum(x2t, axis=1, keepdims=True)
        st_ref[:, 1:2] += jnp.sum(x2t * x2t, axis=1, keepdims=True)

    x2, stats2 = pl.pallas_call(
        kb,
        grid=(B, NT),
        in_specs=[
            pl.BlockSpec((1, TQ, OC1), lambda b, t: (b, t, 0)),
            pl.BlockSpec((1, OC1), lambda b, t: (0, 0)),
            pl.BlockSpec((1, OC1), lambda b, t: (0, 0)),
            pl.BlockSpec((OC2, OC1), lambda b, t: (0, 0)),
        ],
        out_specs=[
            pl.BlockSpec((1, OC2, TQ), lambda b, t: (b, 0, t)),
            pl.BlockSpec((OC2, 2), lambda b, t: (0, 0)),
        ],
        out_shape=[
            jax.ShapeDtypeStruct((B, OC2, N1), jnp.bfloat16),
            jax.ShapeDtypeStruct((OC2, 2), jnp.float32),
        ],
    )(x1, sc1, sh1, W2b)

    mean2 = stats2[:, 0] / NPTS
    var2 = stats2[:, 1] / NPTS - mean2 * mean2
    rstd2 = g2 / jnp.sqrt(var2 + _EPS)
    sc2 = rstd2.reshape(OC2, 1)
    sh2 = (be2 - mean2 * rstd2).reshape(OC2, 1)

    def kc(x2_ref, sc_ref, sh_ref, o_ref):
        o_ref[0] = jnp.maximum(
            x2_ref[0].astype(jnp.float32) * sc_ref[...] + sh_ref[...], 0.0)

    out = pl.pallas_call(
        kc,
        grid=(B, NT),
        in_specs=[
            pl.BlockSpec((1, OC2, TQ), lambda b, t: (b, 0, t)),
            pl.BlockSpec((OC2, 1), lambda b, t: (0, 0)),
            pl.BlockSpec((OC2, 1), lambda b, t: (0, 0)),
        ],
        out_specs=pl.BlockSpec((1, OC2, TQ), lambda b, t: (b, 0, t)),
        out_shape=jax.ShapeDtypeStruct((B, OC2, N1), jnp.float32),
    )(x2, sc2, sh2)

    return out
